# baseline (device time: 44050 ns/iter reference)
import functools

import jax
import jax.numpy as jnp
from jax import lax
from jax.experimental import pallas as pl
from jax.experimental.pallas import tpu as pltpu

N_DEV = 16
N_DIMS = 4
N_LAYERS = 3
N_Q = 2
N_SLOTS = N_LAYERS * N_DIMS * N_Q


def _partner_idx(my_idx, dim):
    z = my_idx // 4
    p = my_idx % 4
    g = p ^ (p >> 1)
    x = g & 1
    y = g >> 1
    if dim == 0:
        x = 1 - x
    elif dim == 1:
        y = 1 - y
    elif dim == 2:
        z = z ^ 1
    else:
        z = z ^ 2
    g2 = y * 2 + x
    return z * 4 + (g2 ^ (g2 >> 1))


def kernel(x, Win0, Wout0, Win1, Wout1, Win2, Wout2):
    b, d_shard = x.shape
    h_dim = Win0.shape[1]
    bq = b // N_Q

    def body(x_ref, wi0, wo0, wi1, wo1, wi2, wo2, out_ref,
             send_buf, recv_buf, send_sems, recv_sems):
        my = lax.axis_index("i")
        partners = [_partner_idx(my, d) for d in range(N_DIMS)]

        barrier = pltpu.get_barrier_semaphore()
        for d in range(N_DIMS):
            pl.semaphore_signal(
                barrier, inc=1,
                device_id=(partners[d],),
                device_id_type=pl.DeviceIdType.MESH,
            )
        pl.semaphore_wait(barrier, N_DIMS)

        def make_rdma(l, s, q):
            slot = (l * N_DIMS + s) * N_Q + q
            dim = (s + q * (N_DIMS // N_Q)) % N_DIMS
            return slot, pltpu.make_async_remote_copy(
                src_ref=send_buf.at[slot],
                dst_ref=recv_buf.at[slot],
                send_sem=send_sems.at[slot],
                recv_sem=recv_sems.at[slot],
                device_id=(partners[dim],),
                device_id_type=pl.DeviceIdType.MESH,
            )

        x_val = x_ref[...]
        for l, (wi, wo) in enumerate([(wi0, wo0), (wi1, wo1), (wi2, wo2)]):
            partial = jnp.dot(x_val, wi[...], preferred_element_type=jnp.float32)
            acc = [partial[q * bq:(q + 1) * bq, :] for q in range(N_Q)]
            for q in range(N_Q):
                slot, rdma = make_rdma(l, 0, q)
                send_buf[slot, :, :] = acc[q]
                rdma.start()
            for s in range(N_DIMS):
                for q in range(N_Q):
                    slot, rdma = make_rdma(l, s, q)
                    rdma.wait()
                    acc[q] = acc[q] + recv_buf[slot, :, :]
                    if s + 1 < N_DIMS:
                        nslot, nrdma = make_rdma(l, s + 1, q)
                        send_buf[nslot, :, :] = acc[q]
                        nrdma.start()
            h = jnp.maximum(jnp.concatenate(acc, axis=0), 0.0)
            x_val = jnp.dot(h, wo[...], preferred_element_type=jnp.float32)
        out_ref[...] = x_val


    return pl.pallas_call(
        body,
        out_shape=jax.ShapeDtypeStruct((b, d_shard), jnp.float32),
        in_specs=[pl.BlockSpec(memory_space=pltpu.VMEM)] * 7,
        out_specs=pl.BlockSpec(memory_space=pltpu.VMEM),
        scratch_shapes=[
            pltpu.VMEM((N_SLOTS, bq, h_dim), jnp.float32),
            pltpu.VMEM((N_SLOTS, bq, h_dim), jnp.float32),
            pltpu.SemaphoreType.DMA((N_SLOTS,)),
            pltpu.SemaphoreType.DMA((N_SLOTS,)),
        ],
        compiler_params=pltpu.CompilerParams(collective_id=0),
    )(x, Win0, Wout0, Win1, Wout1, Win2, Wout2)


# device time: 39990 ns/iter; 1.1015x vs baseline; 1.1015x over previous
import functools

import jax
import jax.numpy as jnp
from jax import lax
from jax.experimental import pallas as pl
from jax.experimental.pallas import tpu as pltpu

N_DEV = 16
N_DIMS = 4
N_LAYERS = 3
N_Q = 4
N_SLOTS = N_LAYERS * N_DIMS * N_Q


def _partner_idx(my_idx, dim):
    z = my_idx // 4
    p = my_idx % 4
    g = p ^ (p >> 1)
    x = g & 1
    y = g >> 1
    if dim == 0:
        x = 1 - x
    elif dim == 1:
        y = 1 - y
    elif dim == 2:
        z = z ^ 1
    else:
        z = z ^ 2
    g2 = y * 2 + x
    return z * 4 + (g2 ^ (g2 >> 1))


def kernel(x, Win0, Wout0, Win1, Wout1, Win2, Wout2):
    b, d_shard = x.shape
    h_dim = Win0.shape[1]
    bq = b // N_Q

    def body(x_ref, wi0, wo0, wi1, wo1, wi2, wo2, out_ref,
             send_buf, recv_buf, send_sems, recv_sems):
        my = lax.axis_index("i")
        partners = [_partner_idx(my, d) for d in range(N_DIMS)]

        barrier = pltpu.get_barrier_semaphore()
        for d in range(N_DIMS):
            pl.semaphore_signal(
                barrier, inc=1,
                device_id=(partners[d],),
                device_id_type=pl.DeviceIdType.MESH,
            )
        pl.semaphore_wait(barrier, N_DIMS)

        def make_rdma(l, s, q):
            slot = (l * N_DIMS + s) * N_Q + q
            dim = (s + q * (N_DIMS // N_Q)) % N_DIMS
            return slot, pltpu.make_async_remote_copy(
                src_ref=send_buf.at[slot],
                dst_ref=recv_buf.at[slot],
                send_sem=send_sems.at[slot],
                recv_sem=recv_sems.at[slot],
                device_id=(partners[dim],),
                device_id_type=pl.DeviceIdType.MESH,
            )

        wos = [wo0, wo1, wo2]
        wis = [wi0, wi1, wi2]

        partial = jnp.dot(x_ref[...], wi0[...], preferred_element_type=jnp.float32)
        acc = [partial[q * bq:(q + 1) * bq, :] for q in range(N_Q)]
        for q in range(N_Q):
            slot, rdma = make_rdma(0, 0, q)
            send_buf[slot, :, :] = acc[q]
            rdma.start()
        for l in range(N_LAYERS):
            for s in range(N_DIMS):
                for q in range(N_Q):
                    slot, rdma = make_rdma(l, s, q)
                    rdma.wait()
                    acc[q] = acc[q] + recv_buf[slot, :, :]
                    if s + 1 < N_DIMS:
                        nslot, nrdma = make_rdma(l, s + 1, q)
                        send_buf[nslot, :, :] = acc[q]
                        nrdma.start()
                    else:
                        hq = jnp.maximum(acc[q], 0.0)
                        xq = jnp.dot(hq, wos[l][...],
                                     preferred_element_type=jnp.float32)
                        if l + 1 < N_LAYERS:
                            acc[q] = jnp.dot(xq, wis[l + 1][...],
                                             preferred_element_type=jnp.float32)
                            nslot, nrdma = make_rdma(l + 1, 0, q)
                            send_buf[nslot, :, :] = acc[q]
                            nrdma.start()
                        else:
                            out_ref[q * bq:(q + 1) * bq, :] = xq


    return pl.pallas_call(
        body,
        out_shape=jax.ShapeDtypeStruct((b, d_shard), jnp.float32),
        in_specs=[pl.BlockSpec(memory_space=pltpu.VMEM)] * 7,
        out_specs=pl.BlockSpec(memory_space=pltpu.VMEM),
        scratch_shapes=[
            pltpu.VMEM((N_SLOTS, bq, h_dim), jnp.float32),
            pltpu.VMEM((N_SLOTS, bq, h_dim), jnp.float32),
            pltpu.SemaphoreType.DMA((N_SLOTS,)),
            pltpu.SemaphoreType.DMA((N_SLOTS,)),
        ],
        compiler_params=pltpu.CompilerParams(collective_id=0),
    )(x, Win0, Wout0, Win1, Wout1, Win2, Wout2)


# device time: 33173 ns/iter; 1.3279x vs baseline; 1.2055x over previous
import jax
import jax.numpy as jnp
from jax import lax
from jax.experimental import pallas as pl
from jax.experimental.pallas import tpu as pltpu

N_DEV = 16
N_LAYERS = 3
N_Q = 4
N_GROUPS = 2
N_PEERS = 3
N_SSLOT = N_LAYERS * N_GROUPS * N_Q
N_RSLOT = N_SSLOT * N_PEERS


def _xyz(idx):
    z = idx // 4
    p = idx % 4
    g = p ^ (p >> 1)
    return g & 1, g >> 1, z


def _ring(x, y, z):
    g2 = y * 2 + x
    return z * 4 + (g2 ^ (g2 >> 1))


def kernel(x, Win0, Wout0, Win1, Wout1, Win2, Wout2):
    b, d_shard = x.shape
    h_dim = Win0.shape[1]
    bq = b // N_Q

    def body(x_ref, wi0, wo0, wi1, wo1, wi2, wo2, out_ref,
             send_buf, recv_buf, send_sems, recv_sems):
        my = lax.axis_index("i")
        mx, my_, mz = _xyz(my)
        peers = [
            [_ring(1 - mx, my_, mz),
             _ring(mx, 1 - my_, mz),
             _ring(1 - mx, 1 - my_, mz)],
            [_ring(mx, my_, (mz + 1) % 4),
             _ring(mx, my_, (mz + 2) % 4),
             _ring(mx, my_, (mz + 3) % 4)],
        ]
        dst_j = [[0, 1, 2], [2, 1, 0]]

        barrier = pltpu.get_barrier_semaphore()
        for g in range(N_GROUPS):
            for j in range(N_PEERS):
                pl.semaphore_signal(
                    barrier, inc=1,
                    device_id=(peers[g][j],),
                    device_id_type=pl.DeviceIdType.MESH,
                )
        pl.semaphore_wait(barrier, N_GROUPS * N_PEERS)

        def sslot(l, g, q):
            return (l * N_GROUPS + g) * N_Q + q

        def make_rdmas(l, g, q):
            out = []
            for j in range(N_PEERS):
                out.append(pltpu.make_async_remote_copy(
                    src_ref=send_buf.at[sslot(l, g, q)],
                    dst_ref=recv_buf.at[sslot(l, g, q) * N_PEERS + dst_j[g][j]],
                    send_sem=send_sems.at[sslot(l, g, q) * N_PEERS + j],
                    recv_sem=recv_sems.at[sslot(l, g, q) * N_PEERS + dst_j[g][j]],
                    device_id=(peers[g][j],),
                    device_id_type=pl.DeviceIdType.MESH,
                ))
            return out

        def launch(l, g, q, val):
            send_buf[sslot(l, g, q), :, :] = val
            for r in make_rdmas(l, g, q):
                r.start()

        def wait_and_sum(l, g, q, val):
            for j, r in enumerate(make_rdmas(l, g, q)):
                r.wait()
                val = val + recv_buf[sslot(l, g, q) * N_PEERS + dst_j[g][j], :, :]
            return val

        def order(q):
            return [0, 1] if q % 2 == 0 else [1, 0]

        wos = [wo0, wo1, wo2]
        wis = [wi0, wi1, wi2]

        partial = jnp.dot(x_ref[...], wi0[...], preferred_element_type=jnp.float32)
        acc = [partial[q * bq:(q + 1) * bq, :] for q in range(N_Q)]
        for q in range(N_Q):
            launch(0, order(q)[0], q, acc[q])
        for l in range(N_LAYERS):
            for r in range(N_GROUPS):
                for q in range(N_Q):
                    g = order(q)[r]
                    acc[q] = wait_and_sum(l, g, q, acc[q])
                    if r == 0:
                        launch(l, order(q)[1], q, acc[q])
                    else:
                        hq = jnp.maximum(acc[q], 0.0)
                        xq = jnp.dot(hq, wos[l][...],
                                     preferred_element_type=jnp.float32)
                        if l + 1 < N_LAYERS:
                            acc[q] = jnp.dot(xq, wis[l + 1][...],
                                             preferred_element_type=jnp.float32)
                            launch(l + 1, order(q)[0], q, acc[q])
                        else:
                            out_ref[q * bq:(q + 1) * bq, :] = xq


    return pl.pallas_call(
        body,
        out_shape=jax.ShapeDtypeStruct((b, d_shard), jnp.float32),
        in_specs=[pl.BlockSpec(memory_space=pltpu.VMEM)] * 7,
        out_specs=pl.BlockSpec(memory_space=pltpu.VMEM),
        scratch_shapes=[
            pltpu.VMEM((N_SSLOT, bq, h_dim), jnp.float32),
            pltpu.VMEM((N_RSLOT, bq, h_dim), jnp.float32),
            pltpu.SemaphoreType.DMA((N_RSLOT,)),
            pltpu.SemaphoreType.DMA((N_RSLOT,)),
        ],
        compiler_params=pltpu.CompilerParams(collective_id=0),
    )(x, Win0, Wout0, Win1, Wout1, Win2, Wout2)
